# baseline (speedup 1.0000x reference)
import jax
import jax.numpy as jnp
from jax import lax
from jax.experimental import pallas as pl
from jax.experimental.pallas import tpu as pltpu
from jax.experimental.pallas import tpu_sc as plsc

_HIDDEN = 768
_LANES = 16
_RS = float(_HIDDEN) ** 0.5
_NC, _NS = 2, 16
_NW = _NC * _NS
_C = 16
_TTREP = 128
_NGB = 8
_EPS = 1e-12


def _rsqrt16(v):
    bits = plsc.bitcast(v, jnp.int32)
    bits = jnp.int32(0x5F3759DF) - lax.shift_right_logical(bits, jnp.int32(1))
    y = plsc.bitcast(bits, jnp.float32)
    for _ in range(3):
        y = y * (1.5 - 0.5 * v * y * y)
    return y


def _sc_body(ids_hbm, pos_hbm, tt_hbm, wtab_hbm, ptab_hbm, ttab_hbm,
             gam_hbm, bet_hbm, out_hbm,
             idxw_v, idxp_v, idxt_v,
             wb0, pb0, tb0, wb1, pb1, tb1, ob0, ob1,
             gam_v, bet_v, stat_v,
             gsem0, gsem1, osem0, osem1):
    wid = lax.axis_index("s") * _NC + lax.axis_index("c")
    ntok = out_hbm.shape[0]
    per_w = ntok // _NW
    nch = per_w // _C
    base = wid * per_w
    lanes = lax.iota(jnp.int32, _LANES)
    lane0 = lanes == 0

    pltpu.sync_copy(gam_hbm, gam_v)
    pltpu.sync_copy(bet_hbm, bet_v)
    pltpu.sync_copy(ids_hbm.at[pl.ds(base, per_w)], idxw_v)
    pltpu.sync_copy(pos_hbm.at[pl.ds(base, per_w)], idxp_v)
    pltpu.sync_copy(tt_hbm.at[pl.ds(base, per_w)], idxt_v)

    def spread(g, carry):
        sl = pl.ds(g * _LANES, _LANES)
        rep = (jnp.full((_LANES,), wid * _LANES + g * _LANES, jnp.int32)
               + lanes) & jnp.int32(_TTREP - 1)
        idxt_v[sl] = idxt_v[sl] + lax.shift_left(rep, jnp.int32(1))
        return carry

    lax.fori_loop(0, per_w // _LANES, spread, 0)

    gsets = ((wb0, pb0, tb0, gsem0), (wb1, pb1, tb1, gsem1))
    osets = ((ob0, osem0), (ob1, osem1))

    def fire(ci, s):
        sl = pl.ds(ci * _C, _C)
        wb, pb, tb, gsem = gsets[s]
        pltpu.async_copy(wtab_hbm.at[idxw_v.at[sl]], wb, gsem)
        pltpu.async_copy(ptab_hbm.at[idxp_v.at[sl]], pb, gsem)
        pltpu.async_copy(ttab_hbm.at[idxt_v.at[sl]], tb, gsem)

    def wait_gathers(s):
        wb, pb, tb, gsem = gsets[s]
        pltpu.make_async_copy(wtab_hbm.at[idxw_v.at[pl.ds(0, _C)]], wb,
                              gsem).wait()
        pltpu.make_async_copy(ptab_hbm.at[idxp_v.at[pl.ds(0, _C)]], pb,
                              gsem).wait()
        pltpu.make_async_copy(ttab_hbm.at[idxt_v.at[pl.ds(0, _C)]], tb,
                              gsem).wait()

    def wait_writeback(s):
        ob, osem = osets[s]
        pltpu.make_async_copy(ob, out_hbm.at[pl.ds(0, _C)], osem).wait()

    fire(0, 0)

    def outer(cj, carry):
        for b in range(2):
            ci = 2 * cj + b
            wb, pb, tb, _ = gsets[b]
            ob, osem = osets[b]

            @pl.when(ci + 1 < nch)
            def _():
                fire(ci + 1, 1 - b)

            wait_gathers(b)

            @pl.when(ci >= 2)
            def _():
                wait_writeback(b)

            @plsc.parallel_loop(0, _C, unroll=2)
            def tok_body(t):
                accs = [jnp.zeros((_LANES,), jnp.float32) for _ in range(4)]
                acc2s = [jnp.zeros((_LANES,), jnp.float32) for _ in range(4)]
                for j in range(_HIDDEN // _LANES):
                    sl = pl.ds(j * _LANES, _LANES)
                    h = wb[t, sl] * _RS + pb[t, sl] + tb[t, sl]
                    ob[t, sl] = h
                    accs[j % 4] = accs[j % 4] + h
                    acc2s[j % 4] = acc2s[j % 4] + h * h
                acc = (accs[0] + accs[1]) + (accs[2] + accs[3])
                acc2 = (acc2s[0] + acc2s[1]) + (acc2s[2] + acc2s[3])
                s1 = jnp.full((_LANES,), jnp.sum(acc), jnp.float32)
                s2 = jnp.full((_LANES,), jnp.sum(acc2), jnp.float32)
                tv = jnp.full((_LANES,), t, jnp.int32)
                plsc.store_scatter(stat_v, [jnp.int32(0) * tv, tv], s1,
                                   mask=lane0)
                plsc.store_scatter(stat_v, [jnp.int32(0) * tv + 1, tv], s2,
                                   mask=lane0)

            s1 = stat_v[0, :] * (1.0 / _HIDDEN)
            s2 = stat_v[1, :] * (1.0 / _HIDDEN)
            var = s2 - s1 * s1 + _EPS
            inv = _rsqrt16(var)
            stat_v[2, :] = inv
            stat_v[3, :] = s1 * inv

            for jb in range(_HIDDEN // (_NGB * _LANES)):
                gs = [gam_v[pl.ds((jb * _NGB + k) * _LANES, _LANES)]
                      for k in range(_NGB)]
                bs = [bet_v[pl.ds((jb * _NGB + k) * _LANES, _LANES)]
                      for k in range(_NGB)]

                @plsc.parallel_loop(0, _C, unroll=2)
                def norm_body(t):
                    tv = jnp.full((_LANES,), t, jnp.int32)
                    iv = plsc.load_gather(stat_v, [jnp.int32(0) * tv + 2, tv])
                    sv = plsc.load_gather(stat_v, [jnp.int32(0) * tv + 3, tv])
                    for k in range(_NGB):
                        sl = pl.ds((jb * _NGB + k) * _LANES, _LANES)
                        h = ob[t, sl]
                        ob[t, sl] = (h * iv - sv) * gs[k] + bs[k]

            pltpu.async_copy(ob, out_hbm.at[pl.ds(base + ci * _C, _C)], osem)
        return carry

    lax.fori_loop(0, nch // 2, outer, 0)
    wait_writeback(0)
    wait_writeback(1)


@jax.jit
def kernel(input_ids, token_type_ids, position_ids, attention_mask,
           word_embeddings, position_embeddings, token_type_embeddings,
           ln_scale, ln_bias):
    del attention_mask
    b, s = input_ids.shape
    ntok = b * s
    per_w = ntok // _NW
    ids = input_ids.astype(jnp.int32).reshape(ntok)
    pos = position_ids.astype(jnp.int32).reshape(ntok)
    tt = token_type_ids.astype(jnp.int32).reshape(ntok)
    ttab_rep = jnp.tile(token_type_embeddings, (_TTREP, 1))

    mesh = plsc.VectorSubcoreMesh(core_axis_name="c", subcore_axis_name="s",
                                  num_cores=_NC, num_subcores=_NS)
    row = lambda: pltpu.VMEM((_C, _HIDDEN), jnp.float32)
    run = pl.kernel(
        _sc_body,
        out_type=jax.ShapeDtypeStruct((ntok, _HIDDEN), jnp.float32),
        mesh=mesh,
        compiler_params=pltpu.CompilerParams(needs_layout_passes=False),
        scratch_types=[
            pltpu.VMEM((per_w,), jnp.int32),
            pltpu.VMEM((per_w,), jnp.int32),
            pltpu.VMEM((per_w,), jnp.int32),
            row(), row(), row(), row(), row(), row(), row(), row(),
            pltpu.VMEM((_HIDDEN,), jnp.float32),
            pltpu.VMEM((_HIDDEN,), jnp.float32),
            pltpu.VMEM((4, _LANES), jnp.float32),
            pltpu.SemaphoreType.DMA,
            pltpu.SemaphoreType.DMA,
            pltpu.SemaphoreType.DMA,
            pltpu.SemaphoreType.DMA,
        ],
    )
    out = run(ids, pos, tt, word_embeddings, position_embeddings,
              ttab_rep, ln_scale, ln_bias)
    return out.reshape(b, s, _HIDDEN)


# speedup vs baseline: 1.4316x; 1.4316x over previous
import jax
import jax.numpy as jnp
from jax import lax
from jax.experimental import pallas as pl
from jax.experimental.pallas import tpu as pltpu
from jax.experimental.pallas import tpu_sc as plsc

_HIDDEN = 768
_LANES = 16
_RS = float(_HIDDEN) ** 0.5
_NC, _NS = 2, 16
_NW = _NC * _NS
_C = 16
_TTREP = 128
_NGB = 16
_EPS = 1e-12


def _rsqrt16(v):
    bits = plsc.bitcast(v, jnp.int32)
    bits = jnp.int32(0x5F3759DF) - lax.shift_right_logical(bits, jnp.int32(1))
    y = plsc.bitcast(bits, jnp.float32)
    for _ in range(3):
        y = y * (1.5 - 0.5 * v * y * y)
    return y


def _sc_body(ids_hbm, pos_hbm, tt_hbm, wtab_hbm, ptab_hbm, ttab_hbm,
             gam_hbm, bet_hbm, out_hbm,
             idxw_v, idxp_v, idxt_v,
             wb0, pb0, tb0, wb1, pb1, tb1, ob0, ob1,
             gam_v, bet_v, stat_v,
             gsem0, gsem1, osem0, osem1):
    wid = lax.axis_index("s") * _NC + lax.axis_index("c")
    ntok = out_hbm.shape[0]
    per_w = ntok // _NW
    nch = per_w // _C
    base = wid * per_w
    lanes = lax.iota(jnp.int32, _LANES)
    lane0 = lanes == 0

    pltpu.sync_copy(gam_hbm, gam_v)
    pltpu.sync_copy(bet_hbm, bet_v)
    pltpu.sync_copy(ids_hbm.at[pl.ds(base, per_w)], idxw_v)
    pltpu.sync_copy(pos_hbm.at[pl.ds(base, per_w)], idxp_v)
    pltpu.sync_copy(tt_hbm.at[pl.ds(base, per_w)], idxt_v)

    def spread(g, carry):
        sl = pl.ds(g * _LANES, _LANES)
        rep = (jnp.full((_LANES,), wid * _LANES + g * _LANES, jnp.int32)
               + lanes) & jnp.int32(_TTREP - 1)
        idxt_v[sl] = idxt_v[sl] + lax.shift_left(rep, jnp.int32(1))
        return carry

    lax.fori_loop(0, per_w // _LANES, spread, 0)

    gsets = ((wb0, pb0, tb0, gsem0), (wb1, pb1, tb1, gsem1))
    osets = ((ob0, osem0), (ob1, osem1))

    def fire(ci, s):
        sl = pl.ds(ci * _C, _C)
        wb, pb, tb, gsem = gsets[s]
        pltpu.async_copy(wtab_hbm.at[idxw_v.at[sl]], wb, gsem)
        pltpu.async_copy(ptab_hbm.at[idxp_v.at[sl]], pb, gsem)
        pltpu.async_copy(ttab_hbm.at[idxt_v.at[sl]], tb, gsem)

    def wait_gathers(s):
        wb, pb, tb, gsem = gsets[s]
        pltpu.make_async_copy(wtab_hbm.at[idxw_v.at[pl.ds(0, _C)]], wb,
                              gsem).wait()
        pltpu.make_async_copy(ptab_hbm.at[idxp_v.at[pl.ds(0, _C)]], pb,
                              gsem).wait()
        pltpu.make_async_copy(ttab_hbm.at[idxt_v.at[pl.ds(0, _C)]], tb,
                              gsem).wait()

    def wait_writeback(s):
        ob, osem = osets[s]
        pltpu.make_async_copy(ob, out_hbm.at[pl.ds(0, _C)], osem).wait()

    fire(0, 0)

    def outer(cj, carry):
        for b in range(2):
            ci = 2 * cj + b
            wb, pb, tb, _ = gsets[b]
            ob, osem = osets[b]

            @pl.when(ci + 1 < nch)
            def _():
                fire(ci + 1, 1 - b)

            wait_gathers(b)

            @pl.when(ci >= 2)
            def _():
                wait_writeback(b)

            @plsc.parallel_loop(0, _C, unroll=4)
            def tok_body(t):
                accs = [jnp.zeros((_LANES,), jnp.float32) for _ in range(4)]
                acc2s = [jnp.zeros((_LANES,), jnp.float32) for _ in range(4)]
                for j in range(_HIDDEN // _LANES):
                    sl = pl.ds(j * _LANES, _LANES)
                    h = wb[t, sl] * _RS + pb[t, sl] + tb[t, sl]
                    ob[t, sl] = h
                    accs[j % 4] = accs[j % 4] + h
                    acc2s[j % 4] = acc2s[j % 4] + h * h
                acc = (accs[0] + accs[1]) + (accs[2] + accs[3])
                acc2 = (acc2s[0] + acc2s[1]) + (acc2s[2] + acc2s[3])
                s1 = jnp.full((_LANES,), jnp.sum(acc), jnp.float32)
                s2 = jnp.full((_LANES,), jnp.sum(acc2), jnp.float32)
                tv = jnp.full((_LANES,), t, jnp.int32)
                plsc.store_scatter(stat_v, [jnp.int32(0) * tv, tv], s1,
                                   mask=lane0)
                plsc.store_scatter(stat_v, [jnp.int32(0) * tv + 1, tv], s2,
                                   mask=lane0)

            s1 = stat_v[0, :] * (1.0 / _HIDDEN)
            s2 = stat_v[1, :] * (1.0 / _HIDDEN)
            var = s2 - s1 * s1 + _EPS
            inv = _rsqrt16(var)
            stat_v[2, :] = inv
            stat_v[3, :] = s1 * inv

            for jb in range(_HIDDEN // (_NGB * _LANES)):
                gs = [gam_v[pl.ds((jb * _NGB + k) * _LANES, _LANES)]
                      for k in range(_NGB)]
                bs = [bet_v[pl.ds((jb * _NGB + k) * _LANES, _LANES)]
                      for k in range(_NGB)]

                @plsc.parallel_loop(0, _C, unroll=2)
                def norm_body(t):
                    tv = jnp.full((_LANES,), t, jnp.int32)
                    iv = plsc.load_gather(stat_v, [jnp.int32(0) * tv + 2, tv])
                    sv = plsc.load_gather(stat_v, [jnp.int32(0) * tv + 3, tv])
                    for k in range(_NGB):
                        sl = pl.ds((jb * _NGB + k) * _LANES, _LANES)
                        h = ob[t, sl]
                        ob[t, sl] = (h * iv - sv) * gs[k] + bs[k]

            pltpu.async_copy(ob, out_hbm.at[pl.ds(base + ci * _C, _C)], osem)
        return carry

    lax.fori_loop(0, nch // 2, outer, 0)
    wait_writeback(0)
    wait_writeback(1)


@jax.jit
def kernel(input_ids, token_type_ids, position_ids, attention_mask,
           word_embeddings, position_embeddings, token_type_embeddings,
           ln_scale, ln_bias):
    del attention_mask
    b, s = input_ids.shape
    ntok = b * s
    per_w = ntok // _NW
    ids = input_ids.astype(jnp.int32).reshape(ntok)
    pos = position_ids.astype(jnp.int32).reshape(ntok)
    tt = token_type_ids.astype(jnp.int32).reshape(ntok)
    ttab_rep = jnp.tile(token_type_embeddings, (_TTREP, 1))

    mesh = plsc.VectorSubcoreMesh(core_axis_name="c", subcore_axis_name="s",
                                  num_cores=_NC, num_subcores=_NS)
    row = lambda: pltpu.VMEM((_C, _HIDDEN), jnp.float32)
    run = pl.kernel(
        _sc_body,
        out_type=jax.ShapeDtypeStruct((ntok, _HIDDEN), jnp.float32),
        mesh=mesh,
        compiler_params=pltpu.CompilerParams(needs_layout_passes=False),
        scratch_types=[
            pltpu.VMEM((per_w,), jnp.int32),
            pltpu.VMEM((per_w,), jnp.int32),
            pltpu.VMEM((per_w,), jnp.int32),
            row(), row(), row(), row(), row(), row(), row(), row(),
            pltpu.VMEM((_HIDDEN,), jnp.float32),
            pltpu.VMEM((_HIDDEN,), jnp.float32),
            pltpu.VMEM((4, _LANES), jnp.float32),
            pltpu.SemaphoreType.DMA,
            pltpu.SemaphoreType.DMA,
            pltpu.SemaphoreType.DMA,
            pltpu.SemaphoreType.DMA,
        ],
    )
    out = run(ids, pos, tt, word_embeddings, position_embeddings,
              ttab_rep, ln_scale, ln_bias)
    return out.reshape(b, s, _HIDDEN)
